# baseline (speedup 1.0000x reference)
import functools

import jax
import jax.numpy as jnp
from jax import lax
from jax.experimental import pallas as pl
from jax.experimental.pallas import tpu as pltpu
from jax.experimental.pallas import tpu_sc as plsc

_CODES = 1024
_DIM = 64
_ROWS = 18432
_BLK = 576
_NBLK = _ROWS // _BLK
_LOSS_SCALE = 1.25 / float(_ROWS * _DIM)

_NW = 32
_BPW = _ROWS // _NW
_ICH = 72
_NCH = _BPW // _ICH


def _tc_body(z_ref, emb_ref, codes_ref, loss_ref):
    i = pl.program_id(0)
    z = z_ref[0]
    emb = emb_ref[...]
    x2 = jnp.sum(z * z, axis=1, keepdims=True)
    e2 = jnp.sum(emb * emb, axis=1)
    dot = lax.dot_general(z, emb, (((1,), (1,)), ((), ())),
                          preferred_element_type=jnp.float32)
    dists = (x2 + e2[None, :]) - 2.0 * dot
    mind = jnp.min(dists, axis=1, keepdims=True)
    iota = lax.broadcasted_iota(jnp.int32, dists.shape, 1)
    code = jnp.min(jnp.where(dists == mind, iota, _CODES), axis=1)
    codes_ref[...] = code.reshape(1, _NCH, _ICH)

    @pl.when(i == 0)
    def _init():
        loss_ref[...] = jnp.zeros_like(loss_ref)

    part = jnp.sum(mind)
    loss_ref[...] = loss_ref[...] + jnp.broadcast_to(part, (1, 1, 128))

    @pl.when(i == _NBLK - 1)
    def _scale():
        loss_ref[...] = loss_ref[...] * _LOSS_SCALE


def _tc_call(flat, emb):
    return pl.pallas_call(
        _tc_body,
        grid=(_NBLK,),
        in_specs=[
            pl.BlockSpec((1, _BLK, _DIM), lambda i: (i, 0, 0)),
            pl.BlockSpec((_CODES, _DIM), lambda i: (0, 0)),
        ],
        out_specs=[
            pl.BlockSpec((1, _NCH, _ICH), lambda i: (i, 0, 0)),
            pl.BlockSpec((1, 1, 128), lambda i: (0, 0, 0)),
        ],
        out_shape=[
            jax.ShapeDtypeStruct((_NBLK, _NCH, _ICH), jnp.int32),
            jax.ShapeDtypeStruct((1, 1, 128), jnp.float32),
        ],
    )(flat, emb)


def _sc_gather_body(emb_hbm, idx_hbm, out_hbm, idx_v, rows_v, sem):
    wid = lax.axis_index("s") * 2 + lax.axis_index("c")
    pltpu.sync_copy(idx_hbm.at[wid], idx_v)
    copies = [
        pltpu.async_copy(emb_hbm.at[idx_v.at[j]],
                         rows_v.at[pl.ds(j * _ICH, _ICH)], sem)
        for j in range(_NCH)
    ]
    for c in copies:
        c.wait()
    pltpu.sync_copy(rows_v, out_hbm.at[wid])


@functools.lru_cache(maxsize=1)
def _make_sc_gather():
    return pl.kernel(
        _sc_gather_body,
        mesh=plsc.VectorSubcoreMesh(core_axis_name="c", subcore_axis_name="s"),
        out_type=jax.ShapeDtypeStruct((_NW, _BPW, _DIM), jnp.float32),
        scratch_types=[
            pltpu.VMEM((_NCH, _ICH), jnp.int32),
            pltpu.VMEM((_BPW, _DIM), jnp.float32),
            pltpu.SemaphoreType.DMA,
        ],
        compiler_params=pltpu.CompilerParams(use_tc_tiling_on_sc=False),
    )


def kernel(z_e, emb):
    B, L, D = z_e.shape
    codes3, loss_acc = _tc_call(z_e, emb)
    codes = codes3.reshape(B, L)
    loss = loss_acc[0, 0, 0]
    z_q = _make_sc_gather()(emb, codes3)
    return (z_q, loss, codes)


# speedup vs baseline: 1.0179x; 1.0179x over previous
import functools

import jax
import jax.numpy as jnp
from jax import lax
from jax.experimental import pallas as pl
from jax.experimental.pallas import tpu as pltpu
from jax.experimental.pallas import tpu_sc as plsc

_CODES = 1024
_DIM = 64
_ROWS = 18432
_BLK = 576
_NBLK = _ROWS // _BLK
_LOSS_SCALE = 1.25 / float(_ROWS * _DIM)

_NW = 32
_BPW = _ROWS // _NW
_ICH = 72
_NCH = _BPW // _ICH


def _tc_body(zt_ref, embt_ref, x2_ref, e2_ref, codes_ref, loss_ref):
    i = pl.program_id(0)
    zt = zt_ref[0]
    embt = embt_ref[...]
    x2 = x2_ref[0]
    e2 = e2_ref[...]
    dot = lax.dot_general(embt, zt, (((0,), (0,)), ((), ())),
                          preferred_element_type=jnp.float32)
    dists = (x2 + e2) - 2.0 * dot
    idx = lax.broadcasted_iota(jnp.int32, (_CODES, _BLK), 0)
    v = dists
    k = _CODES // 2
    while k >= 1:
        tv, bv = v[k:2 * k, :], v[0:k, :]
        ti, bi = idx[k:2 * k, :], idx[0:k, :]
        take = (tv < bv) | ((tv == bv) & (ti < bi))
        idx = jnp.where(take, ti, bi)
        v = jnp.minimum(tv, bv)
        k //= 2
    mind = v
    codes_ref[...] = idx.reshape(1, _NCH, _ICH)

    @pl.when(i == 0)
    def _init():
        loss_ref[...] = jnp.zeros_like(loss_ref)

    part = jnp.sum(mind)
    loss_ref[...] = loss_ref[...] + jnp.broadcast_to(part, (1, 1, 128))

    @pl.when(i == _NBLK - 1)
    def _scale():
        loss_ref[...] = loss_ref[...] * _LOSS_SCALE


def _tc_call(zt, embt, x2, e2c):
    return pl.pallas_call(
        _tc_body,
        grid=(_NBLK,),
        in_specs=[
            pl.BlockSpec((1, _DIM, _BLK), lambda i: (i, 0, 0)),
            pl.BlockSpec((_DIM, _CODES), lambda i: (0, 0)),
            pl.BlockSpec((1, 1, _BLK), lambda i: (i, 0, 0)),
            pl.BlockSpec((_CODES, 1), lambda i: (0, 0)),
        ],
        out_specs=[
            pl.BlockSpec((1, _NCH, _ICH), lambda i: (i, 0, 0)),
            pl.BlockSpec((1, 1, 128), lambda i: (0, 0, 0)),
        ],
        out_shape=[
            jax.ShapeDtypeStruct((_NBLK, _NCH, _ICH), jnp.int32),
            jax.ShapeDtypeStruct((1, 1, 128), jnp.float32),
        ],
    )(zt, embt, x2, e2c)


def _sc_gather_body(emb_hbm, idx_hbm, out_hbm, idx_v, rows_v, sem):
    wid = lax.axis_index("s") * 2 + lax.axis_index("c")
    pltpu.sync_copy(idx_hbm.at[wid], idx_v)
    copies = [
        pltpu.async_copy(emb_hbm.at[idx_v.at[j]],
                         rows_v.at[pl.ds(j * _ICH, _ICH)], sem)
        for j in range(_NCH)
    ]
    for c in copies:
        c.wait()
    pltpu.sync_copy(rows_v, out_hbm.at[wid])


@functools.lru_cache(maxsize=1)
def _make_sc_gather():
    return pl.kernel(
        _sc_gather_body,
        mesh=plsc.VectorSubcoreMesh(core_axis_name="c", subcore_axis_name="s"),
        out_type=jax.ShapeDtypeStruct((_NW, _BPW, _DIM), jnp.float32),
        scratch_types=[
            pltpu.VMEM((_NCH, _ICH), jnp.int32),
            pltpu.VMEM((_BPW, _DIM), jnp.float32),
            pltpu.SemaphoreType.DMA,
        ],
        compiler_params=pltpu.CompilerParams(use_tc_tiling_on_sc=False),
    )


def kernel(z_e, emb):
    B, L, D = z_e.shape
    zt = jnp.swapaxes(z_e, 1, 2)
    embt = emb.T
    x2 = jnp.sum(z_e * z_e, axis=2).reshape(B, 1, L)
    e2c = jnp.sum(emb * emb, axis=1).reshape(_CODES, 1)
    codes3, loss_acc = _tc_call(zt, embt, x2, e2c)
    codes = codes3.reshape(B, L)
    loss = loss_acc[0, 0, 0]
    z_q = _make_sc_gather()(emb, codes3)
    return (z_q, loss, codes)


# speedup vs baseline: 1.0510x; 1.0325x over previous
import functools

import jax
import jax.numpy as jnp
from jax import lax
from jax.experimental import pallas as pl
from jax.experimental.pallas import tpu as pltpu
from jax.experimental.pallas import tpu_sc as plsc

_CODES = 1024
_DIM = 64
_ROWS = 18432
_BLK = 576
_NBLK = _ROWS // _BLK
_LOSS_SCALE = 1.25 / float(_ROWS * _DIM)

_NW = 32
_BPW = _ROWS // _NW
_ICH = 72
_NCH = _BPW // _ICH


def _tc_body(zt_ref, embt_ref, x2_ref, e2_ref, codes_ref, loss_ref):
    i = pl.program_id(0)
    zt = zt_ref[0]
    embt = embt_ref[...]
    x2 = x2_ref[0]
    e2 = e2_ref[...]
    dot = lax.dot_general(embt, zt, (((0,), (0,)), ((), ())),
                          preferred_element_type=jnp.float32)
    dists = (x2 + e2) - 2.0 * dot
    idx = lax.broadcasted_iota(jnp.int32, (_CODES, _BLK), 0)
    v = dists
    k = _CODES // 2
    while k >= 1:
        tv, bv = v[k:2 * k, :], v[0:k, :]
        ti, bi = idx[k:2 * k, :], idx[0:k, :]
        take = (tv < bv) | ((tv == bv) & (ti < bi))
        idx = jnp.where(take, ti, bi)
        v = jnp.minimum(tv, bv)
        k //= 2
    mind = v
    codes_ref[...] = idx.reshape(1, _NCH, _ICH)

    @pl.when(i == 0)
    def _init():
        loss_ref[...] = jnp.zeros_like(loss_ref)

    part = jnp.sum(mind)
    loss_ref[...] = loss_ref[...] + jnp.broadcast_to(part, (1, 1, 128))

    @pl.when(i == _NBLK - 1)
    def _scale():
        loss_ref[...] = loss_ref[...] * _LOSS_SCALE


def _tc_call(zt, embt, x2, e2c):
    return pl.pallas_call(
        _tc_body,
        grid=(_NBLK,),
        in_specs=[
            pl.BlockSpec((1, _DIM, _BLK), lambda i: (i, 0, 0)),
            pl.BlockSpec((_DIM, _CODES), lambda i: (0, 0)),
            pl.BlockSpec((1, 1, _BLK), lambda i: (i, 0, 0)),
            pl.BlockSpec((_CODES, 1), lambda i: (0, 0)),
        ],
        out_specs=[
            pl.BlockSpec((1, _NCH, _ICH), lambda i: (i, 0, 0)),
            pl.BlockSpec((1, 1, 128), lambda i: (0, 0, 0)),
        ],
        out_shape=[
            jax.ShapeDtypeStruct((_NBLK, _NCH, _ICH), jnp.int32),
            jax.ShapeDtypeStruct((1, 1, 128), jnp.float32),
        ],
    )(zt, embt, x2, e2c)


def _sc_gather_body(embt_hbm, codes_hbm, out_hbm, emb_v, codes_v, out_v):
    wid = lax.axis_index("s") * 2 + lax.axis_index("c")
    g = lax.rem(wid, 8)
    q = lax.div(wid, 8)
    pltpu.sync_copy(embt_hbm.at[pl.ds(g * 8, 8)], emb_v)
    pltpu.sync_copy(codes_hbm.at[pl.ds(q * 8, 8)], codes_v)

    @pl.loop(0, 8)
    def _bb(bb):
        for gg in range(_BLK // 16):
            cvec = codes_v[bb, pl.ds(gg * 16, 16)]
            for dd in range(8):
                val = plsc.load_gather(
                    emb_v, [jnp.full((16,), dd, jnp.int32), cvec])
                out_v[bb, dd, pl.ds(gg * 16, 16)] = val

    for bb in range(8):
        pltpu.sync_copy(out_v.at[bb],
                        out_hbm.at[q * 8 + bb].at[pl.ds(g * 8, 8)])


@functools.lru_cache(maxsize=1)
def _make_sc_gather():
    return pl.kernel(
        _sc_gather_body,
        mesh=plsc.VectorSubcoreMesh(core_axis_name="c", subcore_axis_name="s"),
        out_type=jax.ShapeDtypeStruct((_NBLK, _DIM, _BLK), jnp.float32),
        scratch_types=[
            pltpu.VMEM((8, _CODES), jnp.float32),
            pltpu.VMEM((8, _BLK), jnp.int32),
            pltpu.VMEM((8, 8, _BLK), jnp.float32),
        ],
        compiler_params=pltpu.CompilerParams(use_tc_tiling_on_sc=False,
                                             needs_layout_passes=False),
    )


def kernel(z_e, emb):
    B, L, D = z_e.shape
    zt = jnp.swapaxes(z_e, 1, 2)
    embt = emb.T
    x2 = jnp.sum(z_e * z_e, axis=2).reshape(B, 1, L)
    e2c = jnp.sum(emb * emb, axis=1).reshape(_CODES, 1)
    codes3, loss_acc = _tc_call(zt, embt, x2, e2c)
    codes = codes3.reshape(B, L)
    loss = loss_acc[0, 0, 0]
    zq_t = _make_sc_gather()(embt, codes)
    return (jnp.swapaxes(zq_t, 1, 2), loss, codes)
